# baseline (speedup 1.0000x reference)
import jax
import jax.numpy as jnp
from jax import lax
from jax.experimental import pallas as pl
from jax.experimental.pallas import tpu as pltpu
from jax.experimental.pallas import tpu_sc as plsc

N = 10000
D = 128
E = 320000
T = 8

NC = 2
NS = 16
NW = NC * NS
K = 128
NSUP = 16
NSUPS = 5
NCHUNK = NSUPS * NSUP
EPT = NCHUNK * K
EPAD = NW * EPT
NROWS = N + 8
RPT = 624
TAIL = N - NS * RPT


def _sc_body(x_hbm, emb_hbm, src_hbm, dst_hbm, typ_hbm, zb_hbm,
             agg_hbm,
             src_v, dst_v, typ_v, rows0_v, rows1_v, acc_sh,
             gsem, esem, ssem):
    c = lax.axis_index("c")
    s = lax.axis_index("s")
    wid = c * NS + s

    r0 = s * RPT
    pltpu.sync_copy(zb_hbm.at[pl.ds(r0, RPT)], acc_sh.at[pl.ds(r0, RPT)])

    @pl.when(s == NS - 1)
    def _init_tail():
        t0 = NS * RPT
        pltpu.sync_copy(zb_hbm.at[pl.ds(t0, TAIL)], acc_sh.at[pl.ds(t0, TAIL)])

    plsc.subcore_barrier()

    rows = (rows0_v, rows1_v)

    def superchunk(u, carry):
        pltpu.sync_copy(src_hbm.at[wid, u], src_v)
        pltpu.sync_copy(dst_hbm.at[wid, u], dst_v)
        pltpu.sync_copy(typ_hbm.at[wid, u], typ_v)

        xg = pltpu.async_copy(x_hbm.at[src_v.at[0]], rows[0], gsem)
        sc_prev = None
        for jj in range(NSUP):
            p = jj & 1
            xg.wait()
            emb = pltpu.async_copy(emb_hbm.at[typ_v.at[jj]], rows[p], esem,
                                   add=True)
            if sc_prev is not None:
                sc_prev.wait()
            if jj + 1 < NSUP:
                xg = pltpu.async_copy(x_hbm.at[src_v.at[jj + 1]],
                                      rows[1 - p], gsem)
            emb.wait()
            sc_prev = pltpu.async_copy(rows[p], acc_sh.at[dst_v.at[jj]],
                                       ssem, add=True)
        sc_prev.wait()
        return carry

    lax.fori_loop(0, NSUPS, superchunk, 0)

    plsc.subcore_barrier()

    pltpu.sync_copy(acc_sh.at[pl.ds(r0, RPT)], agg_hbm.at[c, pl.ds(r0, RPT)])

    @pl.when(s == NS - 1)
    def _write_tail():
        t0 = NS * RPT
        pltpu.sync_copy(acc_sh.at[pl.ds(t0, TAIL)],
                        agg_hbm.at[c, pl.ds(t0, TAIL)])


_sc_aggregate = pl.kernel(
    _sc_body,
    out_type=jax.ShapeDtypeStruct((NC, N, D), jnp.float32),
    mesh=plsc.VectorSubcoreMesh(
        core_axis_name="c", subcore_axis_name="s",
        num_cores=NC, num_subcores=NS,
    ),
    scratch_types=[
        pltpu.VMEM((NSUP, K), jnp.int32),
        pltpu.VMEM((NSUP, K), jnp.int32),
        pltpu.VMEM((NSUP, K), jnp.int32),
        pltpu.VMEM((K, D), jnp.float32),
        pltpu.VMEM((K, D), jnp.float32),
        pltpu.VMEM_SHARED((NROWS, D), jnp.float32),
        pltpu.SemaphoreType.DMA,
        pltpu.SemaphoreType.DMA,
        pltpu.SemaphoreType.DMA,
    ],
)


def _tc_body(x_ref, a0_ref, a1_ref, wst_ref, wmt_ref, bias_ref,
             g_ref, b_ref, o_ref):
    m = a0_ref[...] + a1_ref[...]
    h = (jnp.dot(x_ref[...], wst_ref[...], preferred_element_type=jnp.float32)
         + jnp.dot(m, wmt_ref[...], preferred_element_type=jnp.float32)
         + bias_ref[...])
    h = jnp.maximum(h, 0.0)
    mu = jnp.mean(h, axis=-1, keepdims=True)
    var = jnp.mean((h - mu) * (h - mu), axis=-1, keepdims=True)
    hn = (h - mu) * lax.rsqrt(var + 1e-5)
    o_ref[...] = hn * g_ref[...] + b_ref[...]


_R = 400

_tc_epilogue = pl.pallas_call(
    _tc_body,
    grid=(N // _R,),
    in_specs=[
        pl.BlockSpec((_R, D), lambda i: (i, 0)),
        pl.BlockSpec((_R, D), lambda i: (i, 0)),
        pl.BlockSpec((_R, D), lambda i: (i, 0)),
        pl.BlockSpec((D, D), lambda i: (0, 0)),
        pl.BlockSpec((D, D), lambda i: (0, 0)),
        pl.BlockSpec((1, D), lambda i: (0, 0)),
        pl.BlockSpec((1, D), lambda i: (0, 0)),
        pl.BlockSpec((1, D), lambda i: (0, 0)),
    ],
    out_specs=pl.BlockSpec((_R, D), lambda i: (i, 0)),
    out_shape=jax.ShapeDtypeStruct((N, D), jnp.float32),
)


def kernel(x, edge_index, edge_types, edge_emb, W_self, b_self, W_msg, b_msg,
           ln_gamma, ln_beta):
    pad = EPAD - E
    src = jnp.concatenate(
        [edge_index[0].astype(jnp.int32), jnp.zeros((pad,), jnp.int32)])
    dst = jnp.concatenate(
        [edge_index[1].astype(jnp.int32), jnp.full((pad,), N, jnp.int32)])
    typ = jnp.concatenate(
        [edge_types.astype(jnp.int32), jnp.zeros((pad,), jnp.int32)])
    src = src.reshape(NW, NSUPS, NSUP, K)
    dst = dst.reshape(NW, NSUPS, NSUP, K)
    typ = typ.reshape(NW, NSUPS, NSUP, K)
    zb = jnp.zeros((N, D), jnp.float32)

    agg = _sc_aggregate(x, edge_emb, src, dst, typ, zb)

    bias = (b_self + b_msg).reshape(1, D)
    return _tc_epilogue(
        x, agg[0], agg[1],
        W_self.T, W_msg.T, bias,
        ln_gamma.reshape(1, D), ln_beta.reshape(1, D),
    )


# speedup vs baseline: 1.0001x; 1.0001x over previous
import jax
import jax.numpy as jnp
from jax import lax
from jax.experimental import pallas as pl
from jax.experimental.pallas import tpu as pltpu
from jax.experimental.pallas import tpu_sc as plsc

N = 10000
D = 128
E = 320000
T = 8

NC = 2
NS = 16
NW = NC * NS
K = 128
NSUP = 16
NSUPS = 5
NCHUNK = NSUPS * NSUP
EPT = NCHUNK * K
EPAD = NW * EPT
NDUMMY = 512
NROWS = N + NDUMMY
RPT = 624
TAIL = N - NS * RPT


def _sc_body(x_hbm, emb_hbm, src_hbm, dst_hbm, typ_hbm, zb_hbm,
             agg_hbm,
             src_v, dst_v, typ_v, rows0_v, rows1_v, acc_sh,
             gsem, esem, ssem):
    c = lax.axis_index("c")
    s = lax.axis_index("s")
    wid = c * NS + s

    r0 = s * RPT
    pltpu.sync_copy(zb_hbm.at[pl.ds(r0, RPT)], acc_sh.at[pl.ds(r0, RPT)])

    @pl.when(s == NS - 1)
    def _init_tail():
        t0 = NS * RPT
        pltpu.sync_copy(zb_hbm.at[pl.ds(t0, TAIL)], acc_sh.at[pl.ds(t0, TAIL)])

    plsc.subcore_barrier()

    rows = (rows0_v, rows1_v)

    def superchunk(u, carry):
        pltpu.sync_copy(src_hbm.at[wid, u], src_v)
        pltpu.sync_copy(dst_hbm.at[wid, u], dst_v)
        pltpu.sync_copy(typ_hbm.at[wid, u], typ_v)

        xg = pltpu.async_copy(x_hbm.at[src_v.at[0]], rows[0], gsem)
        sc_prev = None
        for jj in range(NSUP):
            p = jj & 1
            xg.wait()
            emb = pltpu.async_copy(emb_hbm.at[typ_v.at[jj]], rows[p], esem,
                                   add=True)
            if sc_prev is not None:
                sc_prev.wait()
            if jj + 1 < NSUP:
                xg = pltpu.async_copy(x_hbm.at[src_v.at[jj + 1]],
                                      rows[1 - p], gsem)
            emb.wait()
            sc_prev = pltpu.async_copy(rows[p], acc_sh.at[dst_v.at[jj]],
                                       ssem, add=True)
        sc_prev.wait()
        return carry

    lax.fori_loop(0, NSUPS, superchunk, 0)

    plsc.subcore_barrier()

    pltpu.sync_copy(acc_sh.at[pl.ds(r0, RPT)], agg_hbm.at[c, pl.ds(r0, RPT)])

    @pl.when(s == NS - 1)
    def _write_tail():
        t0 = NS * RPT
        pltpu.sync_copy(acc_sh.at[pl.ds(t0, TAIL)],
                        agg_hbm.at[c, pl.ds(t0, TAIL)])


_sc_aggregate = pl.kernel(
    _sc_body,
    out_type=jax.ShapeDtypeStruct((NC, N, D), jnp.float32),
    mesh=plsc.VectorSubcoreMesh(
        core_axis_name="c", subcore_axis_name="s",
        num_cores=NC, num_subcores=NS,
    ),
    scratch_types=[
        pltpu.VMEM((NSUP, K), jnp.int32),
        pltpu.VMEM((NSUP, K), jnp.int32),
        pltpu.VMEM((NSUP, K), jnp.int32),
        pltpu.VMEM((K, D), jnp.float32),
        pltpu.VMEM((K, D), jnp.float32),
        pltpu.VMEM_SHARED((NROWS, D), jnp.float32),
        pltpu.SemaphoreType.DMA,
        pltpu.SemaphoreType.DMA,
        pltpu.SemaphoreType.DMA,
    ],
)


def _tc_body(x_ref, a0_ref, a1_ref, wst_ref, wmt_ref, bias_ref,
             g_ref, b_ref, o_ref):
    m = a0_ref[...] + a1_ref[...]
    h = (jnp.dot(x_ref[...], wst_ref[...], preferred_element_type=jnp.float32)
         + jnp.dot(m, wmt_ref[...], preferred_element_type=jnp.float32)
         + bias_ref[...])
    h = jnp.maximum(h, 0.0)
    mu = jnp.mean(h, axis=-1, keepdims=True)
    var = jnp.mean((h - mu) * (h - mu), axis=-1, keepdims=True)
    hn = (h - mu) * lax.rsqrt(var + 1e-5)
    o_ref[...] = hn * g_ref[...] + b_ref[...]


_R = 400

_tc_epilogue = pl.pallas_call(
    _tc_body,
    grid=(N // _R,),
    in_specs=[
        pl.BlockSpec((_R, D), lambda i: (i, 0)),
        pl.BlockSpec((_R, D), lambda i: (i, 0)),
        pl.BlockSpec((_R, D), lambda i: (i, 0)),
        pl.BlockSpec((D, D), lambda i: (0, 0)),
        pl.BlockSpec((D, D), lambda i: (0, 0)),
        pl.BlockSpec((1, D), lambda i: (0, 0)),
        pl.BlockSpec((1, D), lambda i: (0, 0)),
        pl.BlockSpec((1, D), lambda i: (0, 0)),
    ],
    out_specs=pl.BlockSpec((_R, D), lambda i: (i, 0)),
    out_shape=jax.ShapeDtypeStruct((N, D), jnp.float32),
)


def kernel(x, edge_index, edge_types, edge_emb, W_self, b_self, W_msg, b_msg,
           ln_gamma, ln_beta):
    pad = EPAD - E
    src = jnp.concatenate(
        [edge_index[0].astype(jnp.int32), jnp.zeros((pad,), jnp.int32)])
    dst = jnp.concatenate(
        [edge_index[1].astype(jnp.int32),
         N + (jnp.arange(pad, dtype=jnp.int32) % NDUMMY)])
    typ = jnp.concatenate(
        [edge_types.astype(jnp.int32), jnp.zeros((pad,), jnp.int32)])
    src = src.reshape(NW, NSUPS, NSUP, K)
    dst = dst.reshape(NW, NSUPS, NSUP, K)
    typ = typ.reshape(NW, NSUPS, NSUP, K)
    zb = jnp.zeros((N, D), jnp.float32)

    agg = _sc_aggregate(x, edge_emb, src, dst, typ, zb)

    bias = (b_self + b_msg).reshape(1, D)
    return _tc_epilogue(
        x, agg[0], agg[1],
        W_self.T, W_msg.T, bias,
        ln_gamma.reshape(1, D), ln_beta.reshape(1, D),
    )


# speedup vs baseline: 6.4325x; 6.4319x over previous
import jax
import jax.numpy as jnp
from jax import lax
from jax.experimental import pallas as pl
from jax.experimental.pallas import tpu as pltpu
from jax.experimental.pallas import tpu_sc as plsc

N = 10000
D = 128
E = 320000
T = 8

NC = 2
NS = 16
NW = NC * NS
EPT = E // NW
K = 80
NSUP = 25
NSUPS = EPT // (K * NSUP)
LANES = 16
RPT = 624
TAIL = N - NS * RPT
CPT = 4992
CTAIL = N * T - NS * CPT


def _sc_body(x_hbm, src_hbm, dst_hbm, typ_hbm, zb_hbm, zs_hbm,
             agg_hbm, cnt_hbm,
             src_v, dst_v, typ_v, rows_v, ones_v, cidx_v, acc_sh, cnt_sh,
             sem):
    c = lax.axis_index("c")
    s = lax.axis_index("s")
    wid = c * NS + s

    r0 = s * RPT
    pltpu.sync_copy(zb_hbm.at[pl.ds(r0, RPT)], acc_sh.at[pl.ds(r0, RPT)])
    f0 = s * CPT
    pltpu.sync_copy(zs_hbm.at[pl.ds(f0, CPT)], cnt_sh.at[pl.ds(f0, CPT)])

    @pl.when(s == NS - 1)
    def _init_tail():
        t0 = NS * RPT
        pltpu.sync_copy(zb_hbm.at[pl.ds(t0, TAIL)], acc_sh.at[pl.ds(t0, TAIL)])
        c0 = NS * CPT
        pltpu.sync_copy(zs_hbm.at[pl.ds(c0, CTAIL)],
                        cnt_sh.at[pl.ds(c0, CTAIL)])

    ones16 = jnp.ones((LANES,), jnp.float32)
    for g in range(K // LANES):
        ones_v[pl.ds(g * LANES, LANES)] = ones16

    plsc.subcore_barrier()

    def superchunk(u, carry):
        pltpu.sync_copy(src_hbm.at[wid, u], src_v)
        pltpu.sync_copy(dst_hbm.at[wid, u], dst_v)
        pltpu.sync_copy(typ_hbm.at[wid, u], typ_v)

        def chunk(j, carry2):
            pltpu.async_copy(x_hbm.at[src_v.at[j]], rows_v, sem).wait()
            for g in range(K // LANES):
                d16 = dst_v[j, pl.ds(g * LANES, LANES)]
                t16 = typ_v[j, pl.ds(g * LANES, LANES)]
                cidx_v[0, pl.ds(g * LANES, LANES)] = d16 * T + t16
            pltpu.sync_copy(rows_v, acc_sh.at[dst_v.at[j]], add=True)
            pltpu.sync_copy(ones_v, cnt_sh.at[cidx_v.at[0]], add=True)
            return carry2

        return lax.fori_loop(0, NSUP, chunk, carry)

    lax.fori_loop(0, NSUPS, superchunk, 0)

    plsc.subcore_barrier()

    pltpu.sync_copy(acc_sh.at[pl.ds(r0, RPT)], agg_hbm.at[c, pl.ds(r0, RPT)])
    pltpu.sync_copy(cnt_sh.at[pl.ds(f0, CPT)],
                    cnt_hbm.at[pl.ds(c * N * T + f0, CPT)])

    @pl.when(s == NS - 1)
    def _write_tail():
        t0 = NS * RPT
        pltpu.sync_copy(acc_sh.at[pl.ds(t0, TAIL)],
                        agg_hbm.at[c, pl.ds(t0, TAIL)])
        c0 = NS * CPT
        pltpu.sync_copy(cnt_sh.at[pl.ds(c0, CTAIL)],
                        cnt_hbm.at[pl.ds(c * N * T + c0, CTAIL)])


_sc_aggregate = pl.kernel(
    _sc_body,
    out_type=(
        jax.ShapeDtypeStruct((NC, N, D), jnp.float32),
        jax.ShapeDtypeStruct((NC * N * T,), jnp.float32),
    ),
    mesh=plsc.VectorSubcoreMesh(
        core_axis_name="c", subcore_axis_name="s",
        num_cores=NC, num_subcores=NS,
    ),
    scratch_types=[
        pltpu.VMEM((NSUP, K), jnp.int32),
        pltpu.VMEM((NSUP, K), jnp.int32),
        pltpu.VMEM((NSUP, K), jnp.int32),
        pltpu.VMEM((K, D), jnp.float32),
        pltpu.VMEM((K,), jnp.float32),
        pltpu.VMEM((1, K), jnp.int32),
        pltpu.VMEM_SHARED((N, D), jnp.float32),
        pltpu.VMEM_SHARED((N * T,), jnp.float32),
        pltpu.SemaphoreType.DMA,
    ],
)


def _tc_body(x_ref, a0_ref, a1_ref, c0_ref, c1_ref, emb_ref, wst_ref,
             wmt_ref, bias_ref, g_ref, b_ref, o_ref):
    cnt = c0_ref[...] + c1_ref[...]
    m = a0_ref[...] + a1_ref[...] + jnp.dot(
        cnt, emb_ref[...], preferred_element_type=jnp.float32)
    h = (jnp.dot(x_ref[...], wst_ref[...], preferred_element_type=jnp.float32)
         + jnp.dot(m, wmt_ref[...], preferred_element_type=jnp.float32)
         + bias_ref[...])
    h = jnp.maximum(h, 0.0)
    mu = jnp.mean(h, axis=-1, keepdims=True)
    var = jnp.mean((h - mu) * (h - mu), axis=-1, keepdims=True)
    hn = (h - mu) * lax.rsqrt(var + 1e-5)
    o_ref[...] = hn * g_ref[...] + b_ref[...]


_R = 400

_tc_epilogue = pl.pallas_call(
    _tc_body,
    grid=(N // _R,),
    in_specs=[
        pl.BlockSpec((_R, D), lambda i: (i, 0)),
        pl.BlockSpec((_R, D), lambda i: (i, 0)),
        pl.BlockSpec((_R, D), lambda i: (i, 0)),
        pl.BlockSpec((_R, T), lambda i: (i, 0)),
        pl.BlockSpec((_R, T), lambda i: (i, 0)),
        pl.BlockSpec((T, D), lambda i: (0, 0)),
        pl.BlockSpec((D, D), lambda i: (0, 0)),
        pl.BlockSpec((D, D), lambda i: (0, 0)),
        pl.BlockSpec((1, D), lambda i: (0, 0)),
        pl.BlockSpec((1, D), lambda i: (0, 0)),
        pl.BlockSpec((1, D), lambda i: (0, 0)),
    ],
    out_specs=pl.BlockSpec((_R, D), lambda i: (i, 0)),
    out_shape=jax.ShapeDtypeStruct((N, D), jnp.float32),
)


def kernel(x, edge_index, edge_types, edge_emb, W_self, b_self, W_msg, b_msg,
           ln_gamma, ln_beta):
    src = edge_index[0].reshape(NW, NSUPS, NSUP, K).astype(jnp.int32)
    dst = edge_index[1].reshape(NW, NSUPS, NSUP, K).astype(jnp.int32)
    typ = edge_types.reshape(NW, NSUPS, NSUP, K).astype(jnp.int32)
    zb = jnp.zeros((N, D), jnp.float32)
    zs = jnp.zeros((N * T,), jnp.float32)

    agg, cnt_flat = _sc_aggregate(x, src, dst, typ, zb, zs)
    cnt = cnt_flat.reshape(NC, N, T)

    bias = (b_self + b_msg).reshape(1, D)
    return _tc_epilogue(
        x, agg[0], agg[1], cnt[0], cnt[1], edge_emb,
        W_self.T, W_msg.T, bias,
        ln_gamma.reshape(1, D), ln_beta.reshape(1, D),
    )


# speedup vs baseline: 7.8837x; 1.2256x over previous
import jax
import jax.numpy as jnp
from jax import lax
from jax.experimental import pallas as pl
from jax.experimental.pallas import tpu as pltpu
from jax.experimental.pallas import tpu_sc as plsc

N = 10000
D = 128
E = 320000
T = 8

NC = 2
NS = 16
NW = NC * NS
EPT = E // NW
K = 80
NSUP = 25
NSUPS = EPT // (K * NSUP)
LANES = 16
RPT = 624
TAIL = N - NS * RPT
CPT = 4992
CTAIL = N * T - NS * CPT


def _sc_body(x_hbm, src_hbm, dst_hbm, typ_hbm, zb_hbm, zs_hbm,
             agg_hbm, cnt_hbm,
             src_v, dst_v, typ_v, rows0_v, rows1_v, ones_v, cidx_v, acc_sh,
             cnt_sh, gsem, ssem, csem):
    c = lax.axis_index("c")
    s = lax.axis_index("s")
    wid = c * NS + s

    r0 = s * RPT
    pltpu.sync_copy(zb_hbm.at[pl.ds(r0, RPT)], acc_sh.at[pl.ds(r0, RPT)])
    f0 = s * CPT
    pltpu.sync_copy(zs_hbm.at[pl.ds(f0, CPT)], cnt_sh.at[pl.ds(f0, CPT)])

    @pl.when(s == NS - 1)
    def _init_tail():
        t0 = NS * RPT
        pltpu.sync_copy(zb_hbm.at[pl.ds(t0, TAIL)], acc_sh.at[pl.ds(t0, TAIL)])
        c0 = NS * CPT
        pltpu.sync_copy(zs_hbm.at[pl.ds(c0, CTAIL)],
                        cnt_sh.at[pl.ds(c0, CTAIL)])

    ones16 = jnp.ones((LANES,), jnp.float32)
    for g in range(K // LANES):
        ones_v[pl.ds(g * LANES, LANES)] = ones16

    plsc.subcore_barrier()

    rows = (rows0_v, rows1_v)

    def superchunk(u, carry):
        pltpu.sync_copy(src_hbm.at[wid, u], src_v)
        pltpu.sync_copy(dst_hbm.at[wid, u], dst_v)
        pltpu.sync_copy(typ_hbm.at[wid, u], typ_v)

        xg = pltpu.async_copy(x_hbm.at[src_v.at[0]], rows[0], gsem)
        sc_prev = cs_prev = None
        for jj in range(NSUP):
            p = jj & 1
            xg.wait()
            for g in range(K // LANES):
                d16 = dst_v[jj, pl.ds(g * LANES, LANES)]
                t16 = typ_v[jj, pl.ds(g * LANES, LANES)]
                cidx_v[p, pl.ds(g * LANES, LANES)] = d16 * T + t16
            if sc_prev is not None:
                sc_prev.wait()
                cs_prev.wait()
            if jj + 1 < NSUP:
                xg = pltpu.async_copy(x_hbm.at[src_v.at[jj + 1]],
                                      rows[1 - p], gsem)
            sc_prev = pltpu.async_copy(rows[p], acc_sh.at[dst_v.at[jj]],
                                       ssem, add=True)
            cs_prev = pltpu.async_copy(ones_v, cnt_sh.at[cidx_v.at[p]],
                                       csem, add=True)
        sc_prev.wait()
        cs_prev.wait()
        return carry

    lax.fori_loop(0, NSUPS, superchunk, 0)

    plsc.subcore_barrier()

    pltpu.sync_copy(acc_sh.at[pl.ds(r0, RPT)], agg_hbm.at[c, pl.ds(r0, RPT)])
    pltpu.sync_copy(cnt_sh.at[pl.ds(f0, CPT)],
                    cnt_hbm.at[pl.ds(c * N * T + f0, CPT)])

    @pl.when(s == NS - 1)
    def _write_tail():
        t0 = NS * RPT
        pltpu.sync_copy(acc_sh.at[pl.ds(t0, TAIL)],
                        agg_hbm.at[c, pl.ds(t0, TAIL)])
        c0 = NS * CPT
        pltpu.sync_copy(cnt_sh.at[pl.ds(c0, CTAIL)],
                        cnt_hbm.at[pl.ds(c * N * T + c0, CTAIL)])


_sc_aggregate = pl.kernel(
    _sc_body,
    out_type=(
        jax.ShapeDtypeStruct((NC, N, D), jnp.float32),
        jax.ShapeDtypeStruct((NC * N * T,), jnp.float32),
    ),
    mesh=plsc.VectorSubcoreMesh(
        core_axis_name="c", subcore_axis_name="s",
        num_cores=NC, num_subcores=NS,
    ),
    scratch_types=[
        pltpu.VMEM((NSUP, K), jnp.int32),
        pltpu.VMEM((NSUP, K), jnp.int32),
        pltpu.VMEM((NSUP, K), jnp.int32),
        pltpu.VMEM((K, D), jnp.float32),
        pltpu.VMEM((K, D), jnp.float32),
        pltpu.VMEM((K,), jnp.float32),
        pltpu.VMEM((2, K), jnp.int32),
        pltpu.VMEM_SHARED((N, D), jnp.float32),
        pltpu.VMEM_SHARED((N * T,), jnp.float32),
        pltpu.SemaphoreType.DMA,
        pltpu.SemaphoreType.DMA,
        pltpu.SemaphoreType.DMA,
    ],
)


def _tc_body(x_ref, a0_ref, a1_ref, c0_ref, c1_ref, emb_ref, wst_ref,
             wmt_ref, bias_ref, g_ref, b_ref, o_ref):
    cnt = c0_ref[...] + c1_ref[...]
    m = a0_ref[...] + a1_ref[...] + jnp.dot(
        cnt, emb_ref[...], preferred_element_type=jnp.float32)
    h = (jnp.dot(x_ref[...], wst_ref[...], preferred_element_type=jnp.float32)
         + jnp.dot(m, wmt_ref[...], preferred_element_type=jnp.float32)
         + bias_ref[...])
    h = jnp.maximum(h, 0.0)
    mu = jnp.mean(h, axis=-1, keepdims=True)
    var = jnp.mean((h - mu) * (h - mu), axis=-1, keepdims=True)
    hn = (h - mu) * lax.rsqrt(var + 1e-5)
    o_ref[...] = hn * g_ref[...] + b_ref[...]


_R = 400

_tc_epilogue = pl.pallas_call(
    _tc_body,
    grid=(N // _R,),
    in_specs=[
        pl.BlockSpec((_R, D), lambda i: (i, 0)),
        pl.BlockSpec((_R, D), lambda i: (i, 0)),
        pl.BlockSpec((_R, D), lambda i: (i, 0)),
        pl.BlockSpec((_R, T), lambda i: (i, 0)),
        pl.BlockSpec((_R, T), lambda i: (i, 0)),
        pl.BlockSpec((T, D), lambda i: (0, 0)),
        pl.BlockSpec((D, D), lambda i: (0, 0)),
        pl.BlockSpec((D, D), lambda i: (0, 0)),
        pl.BlockSpec((1, D), lambda i: (0, 0)),
        pl.BlockSpec((1, D), lambda i: (0, 0)),
        pl.BlockSpec((1, D), lambda i: (0, 0)),
    ],
    out_specs=pl.BlockSpec((_R, D), lambda i: (i, 0)),
    out_shape=jax.ShapeDtypeStruct((N, D), jnp.float32),
)


def kernel(x, edge_index, edge_types, edge_emb, W_self, b_self, W_msg, b_msg,
           ln_gamma, ln_beta):
    src = edge_index[0].reshape(NW, NSUPS, NSUP, K).astype(jnp.int32)
    dst = edge_index[1].reshape(NW, NSUPS, NSUP, K).astype(jnp.int32)
    typ = edge_types.reshape(NW, NSUPS, NSUP, K).astype(jnp.int32)
    zb = jnp.zeros((N, D), jnp.float32)
    zs = jnp.zeros((N * T,), jnp.float32)

    agg, cnt_flat = _sc_aggregate(x, src, dst, typ, zb, zs)
    cnt = cnt_flat.reshape(NC, N, T)

    bias = (b_self + b_msg).reshape(1, D)
    return _tc_epilogue(
        x, agg[0], agg[1], cnt[0], cnt[1], edge_emb,
        W_self.T, W_msg.T, bias,
        ln_gamma.reshape(1, D), ln_beta.reshape(1, D),
    )


# speedup vs baseline: 8.0576x; 1.0220x over previous
import jax
import jax.numpy as jnp
from jax import lax
from jax.experimental import pallas as pl
from jax.experimental.pallas import tpu as pltpu
from jax.experimental.pallas import tpu_sc as plsc

N = 10000
D = 128
E = 320000
T = 8

NC = 2
NS = 16
NW = NC * NS
EPT = E // NW
K = 80
NSUP = 25
NSUPS = EPT // (K * NSUP)
LANES = 16
RPT = 624
TAIL = N - NS * RPT
CPT = 4992
CTAIL = N * T - NS * CPT


def _sc_body(x_hbm, src_hbm, dst_hbm, typ_hbm, zb_hbm, zs_hbm,
             agg_hbm, cnt_hbm,
             src_v, dst_v, typ_v, rows0_v, rows1_v, ones_v, cidx_v, acc_sh,
             cnt_sh, gsem, ssem, csem, isem):
    c = lax.axis_index("c")
    s = lax.axis_index("s")
    wid = c * NS + s

    r0 = s * RPT
    pltpu.sync_copy(zb_hbm.at[pl.ds(r0, RPT)], acc_sh.at[pl.ds(r0, RPT)])
    f0 = s * CPT
    pltpu.sync_copy(zs_hbm.at[pl.ds(f0, CPT)], cnt_sh.at[pl.ds(f0, CPT)])

    @pl.when(s == NS - 1)
    def _init_tail():
        t0 = NS * RPT
        pltpu.sync_copy(zb_hbm.at[pl.ds(t0, TAIL)], acc_sh.at[pl.ds(t0, TAIL)])
        c0 = NS * CPT
        pltpu.sync_copy(zs_hbm.at[pl.ds(c0, CTAIL)],
                        cnt_sh.at[pl.ds(c0, CTAIL)])

    ones16 = jnp.ones((LANES,), jnp.float32)
    for g in range(K // LANES):
        ones_v[pl.ds(g * LANES, LANES)] = ones16

    plsc.subcore_barrier()

    rows = (rows0_v, rows1_v)

    pltpu.sync_copy(src_hbm.at[wid, 0], src_v.at[0])
    pltpu.sync_copy(dst_hbm.at[wid, 0], dst_v.at[0])
    pltpu.sync_copy(typ_hbm.at[wid, 0], typ_v.at[0])

    xg = pltpu.async_copy(x_hbm.at[src_v.at[0, 0]], rows[0], gsem)
    sc_prev = cs_prev = None
    idx_pending = ()
    gchunk = 0
    for u in range(NSUPS):
        b = u & 1
        if u + 1 < NSUPS:
            idx_pending = (
                pltpu.async_copy(src_hbm.at[wid, u + 1], src_v.at[1 - b],
                                 isem),
                pltpu.async_copy(dst_hbm.at[wid, u + 1], dst_v.at[1 - b],
                                 isem),
                pltpu.async_copy(typ_hbm.at[wid, u + 1], typ_v.at[1 - b],
                                 isem),
            )
        for jj in range(NSUP):
            p = gchunk & 1
            xg.wait()
            for g in range(K // LANES):
                d16 = dst_v[b, jj, pl.ds(g * LANES, LANES)]
                t16 = typ_v[b, jj, pl.ds(g * LANES, LANES)]
                cidx_v[p, pl.ds(g * LANES, LANES)] = d16 * T + t16
            if sc_prev is not None:
                sc_prev.wait()
                cs_prev.wait()
            if jj + 1 < NSUP:
                xg = pltpu.async_copy(x_hbm.at[src_v.at[b, jj + 1]],
                                      rows[1 - p], gsem)
            elif u + 1 < NSUPS:
                for d in idx_pending:
                    d.wait()
                xg = pltpu.async_copy(x_hbm.at[src_v.at[1 - b, 0]],
                                      rows[1 - p], gsem)
            sc_prev = pltpu.async_copy(rows[p], acc_sh.at[dst_v.at[b, jj]],
                                       ssem, add=True)
            cs_prev = pltpu.async_copy(ones_v, cnt_sh.at[cidx_v.at[p]],
                                       csem, add=True)
            gchunk += 1
    sc_prev.wait()
    cs_prev.wait()

    plsc.subcore_barrier()

    pltpu.sync_copy(acc_sh.at[pl.ds(r0, RPT)], agg_hbm.at[c, pl.ds(r0, RPT)])
    pltpu.sync_copy(cnt_sh.at[pl.ds(f0, CPT)],
                    cnt_hbm.at[pl.ds(c * N * T + f0, CPT)])

    @pl.when(s == NS - 1)
    def _write_tail():
        t0 = NS * RPT
        pltpu.sync_copy(acc_sh.at[pl.ds(t0, TAIL)],
                        agg_hbm.at[c, pl.ds(t0, TAIL)])
        c0 = NS * CPT
        pltpu.sync_copy(cnt_sh.at[pl.ds(c0, CTAIL)],
                        cnt_hbm.at[pl.ds(c * N * T + c0, CTAIL)])


_sc_aggregate = pl.kernel(
    _sc_body,
    out_type=(
        jax.ShapeDtypeStruct((NC, N, D), jnp.float32),
        jax.ShapeDtypeStruct((NC * N * T,), jnp.float32),
    ),
    mesh=plsc.VectorSubcoreMesh(
        core_axis_name="c", subcore_axis_name="s",
        num_cores=NC, num_subcores=NS,
    ),
    scratch_types=[
        pltpu.VMEM((2, NSUP, K), jnp.int32),
        pltpu.VMEM((2, NSUP, K), jnp.int32),
        pltpu.VMEM((2, NSUP, K), jnp.int32),
        pltpu.VMEM((K, D), jnp.float32),
        pltpu.VMEM((K, D), jnp.float32),
        pltpu.VMEM((K,), jnp.float32),
        pltpu.VMEM((2, K), jnp.int32),
        pltpu.VMEM_SHARED((N, D), jnp.float32),
        pltpu.VMEM_SHARED((N * T,), jnp.float32),
        pltpu.SemaphoreType.DMA,
        pltpu.SemaphoreType.DMA,
        pltpu.SemaphoreType.DMA,
        pltpu.SemaphoreType.DMA,
    ],
)


def _tc_body(x_ref, a0_ref, a1_ref, c0_ref, c1_ref, emb_ref, wst_ref,
             wmt_ref, bias_ref, g_ref, b_ref, o_ref):
    cnt = c0_ref[...] + c1_ref[...]
    m = a0_ref[...] + a1_ref[...] + jnp.dot(
        cnt, emb_ref[...], preferred_element_type=jnp.float32)
    h = (jnp.dot(x_ref[...], wst_ref[...], preferred_element_type=jnp.float32)
         + jnp.dot(m, wmt_ref[...], preferred_element_type=jnp.float32)
         + bias_ref[...])
    h = jnp.maximum(h, 0.0)
    mu = jnp.mean(h, axis=-1, keepdims=True)
    var = jnp.mean((h - mu) * (h - mu), axis=-1, keepdims=True)
    hn = (h - mu) * lax.rsqrt(var + 1e-5)
    o_ref[...] = hn * g_ref[...] + b_ref[...]


_R = 400

_tc_epilogue = pl.pallas_call(
    _tc_body,
    grid=(N // _R,),
    in_specs=[
        pl.BlockSpec((_R, D), lambda i: (i, 0)),
        pl.BlockSpec((_R, D), lambda i: (i, 0)),
        pl.BlockSpec((_R, D), lambda i: (i, 0)),
        pl.BlockSpec((_R, T), lambda i: (i, 0)),
        pl.BlockSpec((_R, T), lambda i: (i, 0)),
        pl.BlockSpec((T, D), lambda i: (0, 0)),
        pl.BlockSpec((D, D), lambda i: (0, 0)),
        pl.BlockSpec((D, D), lambda i: (0, 0)),
        pl.BlockSpec((1, D), lambda i: (0, 0)),
        pl.BlockSpec((1, D), lambda i: (0, 0)),
        pl.BlockSpec((1, D), lambda i: (0, 0)),
    ],
    out_specs=pl.BlockSpec((_R, D), lambda i: (i, 0)),
    out_shape=jax.ShapeDtypeStruct((N, D), jnp.float32),
)


def kernel(x, edge_index, edge_types, edge_emb, W_self, b_self, W_msg, b_msg,
           ln_gamma, ln_beta):
    src = edge_index[0].reshape(NW, NSUPS, NSUP, K).astype(jnp.int32)
    dst = edge_index[1].reshape(NW, NSUPS, NSUP, K).astype(jnp.int32)
    typ = edge_types.reshape(NW, NSUPS, NSUP, K).astype(jnp.int32)
    zb = jnp.zeros((N, D), jnp.float32)
    zs = jnp.zeros((N * T,), jnp.float32)

    agg, cnt_flat = _sc_aggregate(x, src, dst, typ, zb, zs)
    cnt = cnt_flat.reshape(NC, N, T)

    bias = (b_self + b_msg).reshape(1, D)
    return _tc_epilogue(
        x, agg[0], agg[1], cnt[0], cnt[1], edge_emb,
        W_self.T, W_msg.T, bias,
        ln_gamma.reshape(1, D), ln_beta.reshape(1, D),
    )


# speedup vs baseline: 8.0912x; 1.0042x over previous
import jax
import jax.numpy as jnp
from jax import lax
from jax.experimental import pallas as pl
from jax.experimental.pallas import tpu as pltpu
from jax.experimental.pallas import tpu_sc as plsc

N = 10000
D = 128
E = 320000
T = 8

NC = 2
NS = 16
NW = NC * NS
EPT = E // NW
K = 80
NSUP = 25
NSUPS = EPT // (K * NSUP)
LANES = 16
RPT = 624
TAIL = N - NS * RPT
CPT = 4992
CTAIL = N * T - NS * CPT


def _sc_body(x_hbm, src_hbm, dst_hbm, typ_hbm, zb_hbm, zs_hbm,
             agg_hbm, cnt_hbm,
             src_v, dst_v, typ_v, rows0_v, rows1_v, ones_v, cidx_v, acc_sh,
             cnt_sh, gsem, ssem, csem, isem):
    c = lax.axis_index("c")
    s = lax.axis_index("s")
    wid = c * NS + s

    r0 = s * RPT
    pltpu.sync_copy(zb_hbm.at[pl.ds(r0, RPT)], acc_sh.at[pl.ds(r0, RPT)])
    f0 = s * CPT
    pltpu.sync_copy(zs_hbm.at[pl.ds(f0, CPT)], cnt_sh.at[pl.ds(f0, CPT)])

    @pl.when(s == NS - 1)
    def _init_tail():
        t0 = NS * RPT
        pltpu.sync_copy(zb_hbm.at[pl.ds(t0, TAIL)], acc_sh.at[pl.ds(t0, TAIL)])
        c0 = NS * CPT
        pltpu.sync_copy(zs_hbm.at[pl.ds(c0, CTAIL)],
                        cnt_sh.at[pl.ds(c0, CTAIL)])

    ones16 = jnp.ones((LANES,), jnp.float32)
    for g in range(K // LANES):
        ones_v[pl.ds(g * LANES, LANES)] = ones16

    plsc.subcore_barrier()

    rows = (rows0_v, rows1_v)

    pltpu.sync_copy(src_hbm.at[wid, 0], src_v.at[0])
    pltpu.sync_copy(dst_hbm.at[wid, 0], dst_v.at[0])
    pltpu.sync_copy(typ_hbm.at[wid, 0], typ_v.at[0])

    xg = pltpu.async_copy(x_hbm.at[src_v.at[0, 0]], rows[0], gsem)
    sc_prev = cs_prev = None
    idx_pending = ()
    gchunk = 0
    for u in range(NSUPS):
        b = u & 1
        if u + 1 < NSUPS:
            idx_pending = (
                pltpu.async_copy(src_hbm.at[wid, u + 1], src_v.at[1 - b],
                                 isem),
                pltpu.async_copy(dst_hbm.at[wid, u + 1], dst_v.at[1 - b],
                                 isem),
                pltpu.async_copy(typ_hbm.at[wid, u + 1], typ_v.at[1 - b],
                                 isem),
            )
        for jj in range(NSUP):
            p = gchunk & 1
            xg.wait()
            for g in range(K // LANES):
                d16 = dst_v[b, jj, pl.ds(g * LANES, LANES)]
                t16 = typ_v[b, jj, pl.ds(g * LANES, LANES)]
                cidx_v[p, pl.ds(g * LANES, LANES)] = d16 * T + t16
            if sc_prev is not None:
                sc_prev.wait()
                cs_prev.wait()
            if jj + 1 < NSUP:
                xg = pltpu.async_copy(x_hbm.at[src_v.at[b, jj + 1]],
                                      rows[1 - p], gsem)
            elif u + 1 < NSUPS:
                for d in idx_pending:
                    d.wait()
                xg = pltpu.async_copy(x_hbm.at[src_v.at[1 - b, 0]],
                                      rows[1 - p], gsem)
            sc_prev = pltpu.async_copy(rows[p], acc_sh.at[dst_v.at[b, jj]],
                                       ssem, add=True)
            cs_prev = pltpu.async_copy(ones_v, cnt_sh.at[cidx_v.at[p]],
                                       csem, add=True)
            gchunk += 1
    sc_prev.wait()
    cs_prev.wait()

    plsc.subcore_barrier()

    pltpu.sync_copy(acc_sh.at[pl.ds(r0, RPT)], agg_hbm.at[c, pl.ds(r0, RPT)])
    pltpu.sync_copy(cnt_sh.at[pl.ds(f0, CPT)],
                    cnt_hbm.at[pl.ds(c * N * T + f0, CPT)])

    @pl.when(s == NS - 1)
    def _write_tail():
        t0 = NS * RPT
        pltpu.sync_copy(acc_sh.at[pl.ds(t0, TAIL)],
                        agg_hbm.at[c, pl.ds(t0, TAIL)])
        c0 = NS * CPT
        pltpu.sync_copy(cnt_sh.at[pl.ds(c0, CTAIL)],
                        cnt_hbm.at[pl.ds(c * N * T + c0, CTAIL)])


_sc_aggregate = pl.kernel(
    _sc_body,
    out_type=(
        jax.ShapeDtypeStruct((NC, N, D), jnp.float32),
        jax.ShapeDtypeStruct((NC * N * T,), jnp.float32),
    ),
    mesh=plsc.VectorSubcoreMesh(
        core_axis_name="c", subcore_axis_name="s",
        num_cores=NC, num_subcores=NS,
    ),
    scratch_types=[
        pltpu.VMEM((2, NSUP, K), jnp.int32),
        pltpu.VMEM((2, NSUP, K), jnp.int32),
        pltpu.VMEM((2, NSUP, K), jnp.int32),
        pltpu.VMEM((K, D), jnp.float32),
        pltpu.VMEM((K, D), jnp.float32),
        pltpu.VMEM((K,), jnp.float32),
        pltpu.VMEM((2, K), jnp.int32),
        pltpu.VMEM_SHARED((N, D), jnp.float32),
        pltpu.VMEM_SHARED((N * T,), jnp.float32),
        pltpu.SemaphoreType.DMA,
        pltpu.SemaphoreType.DMA,
        pltpu.SemaphoreType.DMA,
        pltpu.SemaphoreType.DMA,
    ],
)


def _dot_nt(a, b):
    return lax.dot_general(a, b, (((1,), (1,)), ((), ())),
                           preferred_element_type=jnp.float32)


def _tc_body(x_ref, a0_ref, a1_ref, c0_ref, c1_ref, emb_ref, ws_ref,
             wm_ref, bs_ref, bm_ref, g_ref, b_ref, o_ref):
    cnt = c0_ref[...] + c1_ref[...]
    m = a0_ref[...] + a1_ref[...] + jnp.dot(
        cnt, emb_ref[...], preferred_element_type=jnp.float32)
    h = (_dot_nt(x_ref[...], ws_ref[...]) + _dot_nt(m, wm_ref[...])
         + bs_ref[...] + bm_ref[...])
    h = jnp.maximum(h, 0.0)
    mu = jnp.mean(h, axis=-1, keepdims=True)
    var = jnp.mean((h - mu) * (h - mu), axis=-1, keepdims=True)
    hn = (h - mu) * lax.rsqrt(var + 1e-5)
    o_ref[...] = hn * g_ref[...] + b_ref[...]


_R = 400

_tc_epilogue = pl.pallas_call(
    _tc_body,
    grid=(N // _R,),
    in_specs=[
        pl.BlockSpec((_R, D), lambda i: (i, 0)),
        pl.BlockSpec((_R, D), lambda i: (i, 0)),
        pl.BlockSpec((_R, D), lambda i: (i, 0)),
        pl.BlockSpec((_R, T), lambda i: (i, 0)),
        pl.BlockSpec((_R, T), lambda i: (i, 0)),
        pl.BlockSpec((T, D), lambda i: (0, 0)),
        pl.BlockSpec((D, D), lambda i: (0, 0)),
        pl.BlockSpec((D, D), lambda i: (0, 0)),
        pl.BlockSpec((1, D), lambda i: (0, 0)),
        pl.BlockSpec((1, D), lambda i: (0, 0)),
        pl.BlockSpec((1, D), lambda i: (0, 0)),
        pl.BlockSpec((1, D), lambda i: (0, 0)),
    ],
    out_specs=pl.BlockSpec((_R, D), lambda i: (i, 0)),
    out_shape=jax.ShapeDtypeStruct((N, D), jnp.float32),
)


def kernel(x, edge_index, edge_types, edge_emb, W_self, b_self, W_msg, b_msg,
           ln_gamma, ln_beta):
    src = edge_index[0].reshape(NW, NSUPS, NSUP, K).astype(jnp.int32)
    dst = edge_index[1].reshape(NW, NSUPS, NSUP, K).astype(jnp.int32)
    typ = edge_types.reshape(NW, NSUPS, NSUP, K).astype(jnp.int32)
    zb = jnp.zeros((N, D), jnp.float32)
    zs = jnp.zeros((N * T,), jnp.float32)

    agg, cnt_flat = _sc_aggregate(x, src, dst, typ, zb, zs)
    cnt = cnt_flat.reshape(NC, N, T)

    return _tc_epilogue(
        x, agg[0], agg[1], cnt[0], cnt[1], edge_emb,
        W_self, W_msg, b_self.reshape(1, D), b_msg.reshape(1, D),
        ln_gamma.reshape(1, D), ln_beta.reshape(1, D),
    )


# speedup vs baseline: 10.3028x; 1.2733x over previous
import jax
import jax.numpy as jnp
from jax import lax
from jax.experimental import pallas as pl
from jax.experimental.pallas import tpu as pltpu
from jax.experimental.pallas import tpu_sc as plsc

N = 10000
D = 128
E = 320000
T = 8

NC = 2
NS = 16
NW = NC * NS
EPT = E // NW
K = 80
NSUP = 5
NSUPS = EPT // (K * NSUP)
LANES = 16
RPT = 624
TAIL = N - NS * RPT
CPT = 4992
CTAIL = N * T - NS * CPT


def _sc_body(x_hbm, src_hbm, dst_hbm, typ_hbm, zb_hbm, zs_hbm,
             agg_hbm, cnt_hbm,
             src_v, dst_v, typ_v, rows0_v, rows1_v, rows2_v, ones_v, cidx_v,
             acc_sh, cnt_sh, gsem, ssem, csem, isem):
    c = lax.axis_index("c")
    s = lax.axis_index("s")
    wid = c * NS + s

    r0 = s * RPT
    pltpu.sync_copy(zb_hbm.at[pl.ds(r0, RPT)], acc_sh.at[pl.ds(r0, RPT)])
    f0 = s * CPT
    pltpu.sync_copy(zs_hbm.at[pl.ds(f0, CPT)], cnt_sh.at[pl.ds(f0, CPT)])

    @pl.when(s == NS - 1)
    def _init_tail():
        t0 = NS * RPT
        pltpu.sync_copy(zb_hbm.at[pl.ds(t0, TAIL)], acc_sh.at[pl.ds(t0, TAIL)])
        c0 = NS * CPT
        pltpu.sync_copy(zs_hbm.at[pl.ds(c0, CTAIL)],
                        cnt_sh.at[pl.ds(c0, CTAIL)])

    ones16 = jnp.ones((LANES,), jnp.float32)
    for g in range(K // LANES):
        ones_v[pl.ds(g * LANES, LANES)] = ones16

    plsc.subcore_barrier()

    rows = (rows0_v, rows1_v, rows2_v)
    NB = 3
    TOT = NSUPS * NSUP

    pltpu.sync_copy(src_hbm.at[wid, 0], src_v.at[0])
    pltpu.sync_copy(dst_hbm.at[wid, 0], dst_v.at[0])
    pltpu.sync_copy(typ_hbm.at[wid, 0], typ_v.at[0])

    def chunk_loc(g):
        u = g // NSUP
        return u, u & 1, g % NSUP

    idx_pending = {}
    idx_ready = {0}

    def fire_idx(u):
        if u < NSUPS and u not in idx_pending and u not in idx_ready:
            idx_pending[u] = (
                pltpu.async_copy(src_hbm.at[wid, u], src_v.at[u & 1], isem),
                pltpu.async_copy(dst_hbm.at[wid, u], dst_v.at[u & 1], isem),
                pltpu.async_copy(typ_hbm.at[wid, u], typ_v.at[u & 1], isem),
            )

    def ensure_idx(u):
        if u in idx_pending:
            for d in idx_pending.pop(u):
                d.wait()
            idx_ready.add(u)

    def fire_gather(g):
        u, b, jj = chunk_loc(g)
        ensure_idx(u)
        return pltpu.async_copy(x_hbm.at[src_v.at[b, jj]], rows[g % NB],
                                gsem)

    fire_idx(1)
    xgs = {0: fire_gather(0), 1: fire_gather(1)}
    sc_prev = cs_prev = None
    for g in range(TOT):
        u, b, jj = chunk_loc(g)
        if jj == 0:
            fire_idx(u + 1)
        p = g % NB
        xgs.pop(g).wait()
        for gg in range(K // LANES):
            d16 = dst_v[b, jj, pl.ds(gg * LANES, LANES)]
            t16 = typ_v[b, jj, pl.ds(gg * LANES, LANES)]
            cidx_v[g & 1, pl.ds(gg * LANES, LANES)] = d16 * T + t16
        if sc_prev is not None:
            sc_prev.wait()
            cs_prev.wait()
        if g + 2 < TOT:
            xgs[g + 2] = fire_gather(g + 2)
        sc_prev = pltpu.async_copy(rows[p], acc_sh.at[dst_v.at[b, jj]],
                                   ssem, add=True)
        cs_prev = pltpu.async_copy(ones_v, cnt_sh.at[cidx_v.at[g & 1]],
                                   csem, add=True)
    sc_prev.wait()
    cs_prev.wait()

    plsc.subcore_barrier()

    pltpu.sync_copy(acc_sh.at[pl.ds(r0, RPT)], agg_hbm.at[c, pl.ds(r0, RPT)])
    pltpu.sync_copy(cnt_sh.at[pl.ds(f0, CPT)],
                    cnt_hbm.at[pl.ds(c * N * T + f0, CPT)])

    @pl.when(s == NS - 1)
    def _write_tail():
        t0 = NS * RPT
        pltpu.sync_copy(acc_sh.at[pl.ds(t0, TAIL)],
                        agg_hbm.at[c, pl.ds(t0, TAIL)])
        c0 = NS * CPT
        pltpu.sync_copy(cnt_sh.at[pl.ds(c0, CTAIL)],
                        cnt_hbm.at[pl.ds(c * N * T + c0, CTAIL)])


_sc_aggregate = pl.kernel(
    _sc_body,
    out_type=(
        jax.ShapeDtypeStruct((NC, N, D), jnp.float32),
        jax.ShapeDtypeStruct((NC * N * T,), jnp.float32),
    ),
    mesh=plsc.VectorSubcoreMesh(
        core_axis_name="c", subcore_axis_name="s",
        num_cores=NC, num_subcores=NS,
    ),
    scratch_types=[
        pltpu.VMEM((2, NSUP, K), jnp.int32),
        pltpu.VMEM((2, NSUP, K), jnp.int32),
        pltpu.VMEM((2, NSUP, K), jnp.int32),
        pltpu.VMEM((K, D), jnp.float32),
        pltpu.VMEM((K, D), jnp.float32),
        pltpu.VMEM((K, D), jnp.float32),
        pltpu.VMEM((K,), jnp.float32),
        pltpu.VMEM((2, K), jnp.int32),
        pltpu.VMEM_SHARED((N, D), jnp.float32),
        pltpu.VMEM_SHARED((N * T,), jnp.float32),
        pltpu.SemaphoreType.DMA,
        pltpu.SemaphoreType.DMA,
        pltpu.SemaphoreType.DMA,
        pltpu.SemaphoreType.DMA,
    ],
)


def _dot_nt(a, b):
    return lax.dot_general(a, b, (((1,), (1,)), ((), ())),
                           preferred_element_type=jnp.float32)


def _tc_body(x_ref, a0_ref, a1_ref, c0_ref, c1_ref, emb_ref, ws_ref,
             wm_ref, bs_ref, bm_ref, g_ref, b_ref, o_ref):
    cnt = c0_ref[...] + c1_ref[...]
    m = a0_ref[...] + a1_ref[...] + jnp.dot(
        cnt, emb_ref[...], preferred_element_type=jnp.float32)
    h = (_dot_nt(x_ref[...], ws_ref[...]) + _dot_nt(m, wm_ref[...])
         + bs_ref[...] + bm_ref[...])
    h = jnp.maximum(h, 0.0)
    mu = jnp.mean(h, axis=-1, keepdims=True)
    var = jnp.mean((h - mu) * (h - mu), axis=-1, keepdims=True)
    hn = (h - mu) * lax.rsqrt(var + 1e-5)
    o_ref[...] = hn * g_ref[...] + b_ref[...]


_R = 400

_tc_epilogue = pl.pallas_call(
    _tc_body,
    grid=(N // _R,),
    in_specs=[
        pl.BlockSpec((_R, D), lambda i: (i, 0)),
        pl.BlockSpec((_R, D), lambda i: (i, 0)),
        pl.BlockSpec((_R, D), lambda i: (i, 0)),
        pl.BlockSpec((_R, T), lambda i: (i, 0)),
        pl.BlockSpec((_R, T), lambda i: (i, 0)),
        pl.BlockSpec((T, D), lambda i: (0, 0)),
        pl.BlockSpec((D, D), lambda i: (0, 0)),
        pl.BlockSpec((D, D), lambda i: (0, 0)),
        pl.BlockSpec((1, D), lambda i: (0, 0)),
        pl.BlockSpec((1, D), lambda i: (0, 0)),
        pl.BlockSpec((1, D), lambda i: (0, 0)),
        pl.BlockSpec((1, D), lambda i: (0, 0)),
    ],
    out_specs=pl.BlockSpec((_R, D), lambda i: (i, 0)),
    out_shape=jax.ShapeDtypeStruct((N, D), jnp.float32),
)


def kernel(x, edge_index, edge_types, edge_emb, W_self, b_self, W_msg, b_msg,
           ln_gamma, ln_beta):
    src = edge_index[0].reshape(NW, NSUPS, NSUP, K).astype(jnp.int32)
    dst = edge_index[1].reshape(NW, NSUPS, NSUP, K).astype(jnp.int32)
    typ = edge_types.reshape(NW, NSUPS, NSUP, K).astype(jnp.int32)
    zb = jnp.zeros((N, D), jnp.float32)
    zs = jnp.zeros((N * T,), jnp.float32)

    agg, cnt_flat = _sc_aggregate(x, src, dst, typ, zb, zs)
    cnt = cnt_flat.reshape(NC, N, T)

    return _tc_epilogue(
        x, agg[0], agg[1], cnt[0], cnt[1], edge_emb,
        W_self, W_msg, b_self.reshape(1, D), b_msg.reshape(1, D),
        ln_gamma.reshape(1, D), ln_beta.reshape(1, D),
    )


# speedup vs baseline: 10.5369x; 1.0227x over previous
import jax
import jax.numpy as jnp
from jax import lax
from jax.experimental import pallas as pl
from jax.experimental.pallas import tpu as pltpu
from jax.experimental.pallas import tpu_sc as plsc

N = 10000
D = 128
E = 320000
T = 8

NC = 2
NS = 16
NW = NC * NS
K = 64
NSUP = 5
NSUPS = 32
NCHUNK = NSUPS * NSUP
EPT = NCHUNK * K
EPAD = NW * EPT
NDUMMY = 64
NROWS = N + NDUMMY
LANES = 16
RPT = 624
TAIL = N - NS * RPT
CPT = 4992
CTAIL = N * T - NS * CPT


def _sc_body(x_hbm, src_hbm, dst_hbm, typ_hbm, zb_hbm, zs_hbm,
             agg_hbm, cnt_hbm,
             src_v, dst_v, typ_v, rows0_v, rows1_v, rows2_v, rows3_v, ones_v,
             cidx_v, acc_sh, cnt_sh, gsem, ssem, csem, isem):
    c = lax.axis_index("c")
    s = lax.axis_index("s")
    wid = c * NS + s

    r0 = s * RPT
    pltpu.sync_copy(zb_hbm.at[pl.ds(r0, RPT)], acc_sh.at[pl.ds(r0, RPT)])
    f0 = s * CPT
    pltpu.sync_copy(zs_hbm.at[pl.ds(f0, CPT)], cnt_sh.at[pl.ds(f0, CPT)])

    @pl.when(s == NS - 1)
    def _init_tail():
        t0 = NS * RPT
        pltpu.sync_copy(zb_hbm.at[pl.ds(t0, TAIL)], acc_sh.at[pl.ds(t0, TAIL)])
        c0 = NS * CPT
        pltpu.sync_copy(zs_hbm.at[pl.ds(c0, CTAIL)],
                        cnt_sh.at[pl.ds(c0, CTAIL)])

    ones16 = jnp.ones((LANES,), jnp.float32)
    for g in range(K // LANES):
        ones_v[pl.ds(g * LANES, LANES)] = ones16

    plsc.subcore_barrier()

    rows = (rows0_v, rows1_v, rows2_v, rows3_v)
    NB = 4
    TOT = NSUPS * NSUP

    pltpu.sync_copy(src_hbm.at[wid, 0], src_v.at[0])
    pltpu.sync_copy(dst_hbm.at[wid, 0], dst_v.at[0])
    pltpu.sync_copy(typ_hbm.at[wid, 0], typ_v.at[0])

    def chunk_loc(g):
        u = g // NSUP
        return u, u & 1, g % NSUP

    idx_pending = {}
    idx_ready = {0}

    def fire_idx(u):
        if u < NSUPS and u not in idx_pending and u not in idx_ready:
            idx_pending[u] = (
                pltpu.async_copy(src_hbm.at[wid, u], src_v.at[u & 1], isem),
                pltpu.async_copy(dst_hbm.at[wid, u], dst_v.at[u & 1], isem),
                pltpu.async_copy(typ_hbm.at[wid, u], typ_v.at[u & 1], isem),
            )

    def ensure_idx(u):
        if u in idx_pending:
            for d in idx_pending.pop(u):
                d.wait()
            idx_ready.add(u)

    def fire_gather(g):
        u, b, jj = chunk_loc(g)
        ensure_idx(u)
        return pltpu.async_copy(x_hbm.at[src_v.at[b, jj]], rows[g % NB],
                                gsem)

    fire_idx(1)
    xgs = {0: fire_gather(0), 1: fire_gather(1), 2: fire_gather(2)}
    sc_prev = cs_prev = None
    for g in range(TOT):
        u, b, jj = chunk_loc(g)
        if jj == 0:
            fire_idx(u + 1)
        p = g % NB
        xgs.pop(g).wait()
        for gg in range(K // LANES):
            d16 = dst_v[b, jj, pl.ds(gg * LANES, LANES)]
            t16 = typ_v[b, jj, pl.ds(gg * LANES, LANES)]
            cidx_v[g & 1, pl.ds(gg * LANES, LANES)] = d16 * T + t16
        if sc_prev is not None:
            sc_prev.wait()
            cs_prev.wait()
        if g + 3 < TOT:
            xgs[g + 3] = fire_gather(g + 3)
        sc_prev = pltpu.async_copy(rows[p], acc_sh.at[dst_v.at[b, jj]],
                                   ssem, add=True)
        cs_prev = pltpu.async_copy(ones_v, cnt_sh.at[cidx_v.at[g & 1]],
                                   csem, add=True)
    sc_prev.wait()
    cs_prev.wait()

    plsc.subcore_barrier()

    pltpu.sync_copy(acc_sh.at[pl.ds(r0, RPT)], agg_hbm.at[c, pl.ds(r0, RPT)])
    pltpu.sync_copy(cnt_sh.at[pl.ds(f0, CPT)],
                    cnt_hbm.at[pl.ds(c * N * T + f0, CPT)])

    @pl.when(s == NS - 1)
    def _write_tail():
        t0 = NS * RPT
        pltpu.sync_copy(acc_sh.at[pl.ds(t0, TAIL)],
                        agg_hbm.at[c, pl.ds(t0, TAIL)])
        c0 = NS * CPT
        pltpu.sync_copy(cnt_sh.at[pl.ds(c0, CTAIL)],
                        cnt_hbm.at[pl.ds(c * N * T + c0, CTAIL)])


_sc_aggregate = pl.kernel(
    _sc_body,
    out_type=(
        jax.ShapeDtypeStruct((NC, N, D), jnp.float32),
        jax.ShapeDtypeStruct((NC * N * T,), jnp.float32),
    ),
    mesh=plsc.VectorSubcoreMesh(
        core_axis_name="c", subcore_axis_name="s",
        num_cores=NC, num_subcores=NS,
    ),
    scratch_types=[
        pltpu.VMEM((2, NSUP, K), jnp.int32),
        pltpu.VMEM((2, NSUP, K), jnp.int32),
        pltpu.VMEM((2, NSUP, K), jnp.int32),
        pltpu.VMEM((K, D), jnp.float32),
        pltpu.VMEM((K, D), jnp.float32),
        pltpu.VMEM((K, D), jnp.float32),
        pltpu.VMEM((K, D), jnp.float32),
        pltpu.VMEM((K,), jnp.float32),
        pltpu.VMEM((2, K), jnp.int32),
        pltpu.VMEM_SHARED((NROWS, D), jnp.float32),
        pltpu.VMEM_SHARED((NROWS * T,), jnp.float32),
        pltpu.SemaphoreType.DMA,
        pltpu.SemaphoreType.DMA,
        pltpu.SemaphoreType.DMA,
        pltpu.SemaphoreType.DMA,
    ],
)


def _dot_nt(a, b):
    return lax.dot_general(a, b, (((1,), (1,)), ((), ())),
                           preferred_element_type=jnp.float32)


def _tc_body(x_ref, a0_ref, a1_ref, c0_ref, c1_ref, emb_ref, ws_ref,
             wm_ref, bs_ref, bm_ref, g_ref, b_ref, o_ref):
    cnt = c0_ref[...] + c1_ref[...]
    m = a0_ref[...] + a1_ref[...] + jnp.dot(
        cnt, emb_ref[...], preferred_element_type=jnp.float32)
    h = (_dot_nt(x_ref[...], ws_ref[...]) + _dot_nt(m, wm_ref[...])
         + bs_ref[...] + bm_ref[...])
    h = jnp.maximum(h, 0.0)
    mu = jnp.mean(h, axis=-1, keepdims=True)
    var = jnp.mean((h - mu) * (h - mu), axis=-1, keepdims=True)
    hn = (h - mu) * lax.rsqrt(var + 1e-5)
    o_ref[...] = hn * g_ref[...] + b_ref[...]


_R = 400

_tc_epilogue = pl.pallas_call(
    _tc_body,
    grid=(N // _R,),
    in_specs=[
        pl.BlockSpec((_R, D), lambda i: (i, 0)),
        pl.BlockSpec((_R, D), lambda i: (i, 0)),
        pl.BlockSpec((_R, D), lambda i: (i, 0)),
        pl.BlockSpec((_R, T), lambda i: (i, 0)),
        pl.BlockSpec((_R, T), lambda i: (i, 0)),
        pl.BlockSpec((T, D), lambda i: (0, 0)),
        pl.BlockSpec((D, D), lambda i: (0, 0)),
        pl.BlockSpec((D, D), lambda i: (0, 0)),
        pl.BlockSpec((1, D), lambda i: (0, 0)),
        pl.BlockSpec((1, D), lambda i: (0, 0)),
        pl.BlockSpec((1, D), lambda i: (0, 0)),
        pl.BlockSpec((1, D), lambda i: (0, 0)),
    ],
    out_specs=pl.BlockSpec((_R, D), lambda i: (i, 0)),
    out_shape=jax.ShapeDtypeStruct((N, D), jnp.float32),
)


def kernel(x, edge_index, edge_types, edge_emb, W_self, b_self, W_msg, b_msg,
           ln_gamma, ln_beta):
    pad = EPAD - E
    ar = jnp.arange(pad, dtype=jnp.int32)
    src = jnp.concatenate([edge_index[0].astype(jnp.int32), ar % N])
    dst = jnp.concatenate(
        [edge_index[1].astype(jnp.int32), N + (ar % NDUMMY)])
    typ = jnp.concatenate(
        [edge_types.astype(jnp.int32), jnp.zeros((pad,), jnp.int32)])
    src = src.reshape(NW, NSUPS, NSUP, K)
    dst = dst.reshape(NW, NSUPS, NSUP, K)
    typ = typ.reshape(NW, NSUPS, NSUP, K)
    zb = jnp.zeros((N, D), jnp.float32)
    zs = jnp.zeros((N * T,), jnp.float32)

    agg, cnt_flat = _sc_aggregate(x, src, dst, typ, zb, zs)
    cnt = cnt_flat.reshape(NC, N, T)

    return _tc_epilogue(
        x, agg[0], agg[1], cnt[0], cnt[1], edge_emb,
        W_self, W_msg, b_self.reshape(1, D), b_msg.reshape(1, D),
        ln_gamma.reshape(1, D), ln_beta.reshape(1, D),
    )
